# baseline (speedup 1.0000x reference)
import functools

import jax
import jax.numpy as jnp
from jax import lax
from jax.experimental import pallas as pl
from jax.experimental.pallas import tpu as pltpu
from jax.experimental.pallas import tpu_sc as plsc

_VOCAB = 1000000
_DIM = 64
_BATCH = 16384
_HIST = 50
_N = _BATCH * _HIST
_NW = 32
_PER_W = _N // _NW
_CHUNK = 1024
_NCHUNK = _PER_W // _CHUNK
_SUB = 128
_NSUB = _CHUNK // _SUB
_GROUPS = _CHUNK // 16


def _dequant_group(rows_v, scl_v, out_v, g):
    ids = g * 16 + lax.iota(jnp.int32, 16)
    q4 = ids >> 2
    cb4 = (ids & 3) << 5
    sw = scl_v[pl.ds(g * 16, 16)]
    sdup0 = plsc.bitcast((sw & 0xFFFF) | (sw << 16), jnp.float16)
    sdup1 = plsc.bitcast(lax.shift_right_logical(sw, 16)
                         | (sw & jnp.int32(-65536)), jnp.float16)
    c128 = jnp.full((32,), 128.0, jnp.float16)
    for w in range(16):
        v = plsc.load_gather(rows_v, [ids, jnp.full((16,), w, jnp.int32)])
        vx = v ^ jnp.int32(-2139062144)
        p01 = (vx & 0xFF) | ((vx & 0xFF00) << 8)
        p23 = ((lax.shift_right_logical(vx, 16) & 0xFF)
               | (lax.shift_right_logical(vx, 8) & 0xFF0000))
        s = sdup0 if w < 8 else sdup1
        f01 = (plsc.bitcast(p01, jnp.int16).astype(jnp.float16) - c128) * s
        f23 = (plsc.bitcast(p23, jnp.int16).astype(jnp.float16) - c128) * s
        plsc.store_scatter(out_v, [q4, cb4 + 2 * w],
                           plsc.bitcast(f01, jnp.int32))
        plsc.store_scatter(out_v, [q4, cb4 + 2 * w + 1],
                           plsc.bitcast(f23, jnp.int32))


def _sc_body(idx_hbm, w_hbm, s_hbm, out_hbm, idx_v, rows_v, scl_v, out_v,
             sem_w, sem_s):
    wid = lax.axis_index("s") * 2 + lax.axis_index("c")
    base_row = wid * _PER_W

    def chunk_body(c, _):
        r0 = pl.multiple_of(base_row + c * _CHUNK, _CHUNK)
        pltpu.sync_copy(idx_hbm.at[pl.ds(pl.multiple_of(r0 // _SUB, _NSUB), _NSUB)],
                        idx_v)
        copies = []
        for j in range(_NSUB):
            copies.append(pltpu.async_copy(
                w_hbm.at[idx_v.at[j]], rows_v.at[pl.ds(j * _SUB, _SUB)], sem_w))
            copies.append(pltpu.async_copy(
                s_hbm.at[idx_v.at[j]], scl_v.at[pl.ds(j * _SUB, _SUB)], sem_s))
        for cp in copies:
            cp.wait()

        def group_body(g, _):
            _dequant_group(rows_v, scl_v, out_v, g)
            return 0

        lax.fori_loop(0, _GROUPS, group_body, 0)
        pltpu.sync_copy(out_v,
                        out_hbm.at[pl.ds(pl.multiple_of(r0 // 4, _CHUNK // 4),
                                         _CHUNK // 4)])
        return 0

    lax.fori_loop(0, _NCHUNK, chunk_body, 0)


@functools.partial(
    pl.kernel,
    out_type=jax.ShapeDtypeStruct((_N // 4, 128), jnp.int32),
    mesh=plsc.VectorSubcoreMesh(core_axis_name="c", subcore_axis_name="s"),
    scratch_types=[
        pltpu.VMEM((_NSUB, _SUB), jnp.int32),
        pltpu.VMEM((_CHUNK, 16), jnp.int32),
        pltpu.VMEM((_CHUNK,), jnp.int32),
        pltpu.VMEM((_CHUNK // 4, 128), jnp.int32),
        pltpu.SemaphoreType.DMA,
        pltpu.SemaphoreType.DMA,
    ],
    compiler_params=pltpu.CompilerParams(needs_layout_passes=False,
                                         use_tc_tiling_on_sc=False),
)
def _sc_lookup(idx_hbm, w_hbm, s_hbm, out_hbm, idx_v, rows_v, scl_v, out_v,
               sem_w, sem_s):
    _sc_body(idx_hbm, w_hbm, s_hbm, out_hbm, idx_v, rows_v, scl_v, out_v,
             sem_w, sem_s)


def kernel(indices, weight_int8, scales_fp16):
    V, D = weight_int8.shape
    idx2d = indices.reshape(_N // _SUB, _SUB)
    w_i32 = lax.bitcast_convert_type(weight_int8.reshape(V, D // 4, 4), jnp.int32)
    s_i32 = lax.bitcast_convert_type(scales_fp16, jnp.int32)
    out = _sc_lookup(idx2d, w_i32, s_i32)
    out = lax.bitcast_convert_type(out, jnp.float16)
    return out.reshape(indices.shape + (D,))


# speedup vs baseline: 1.5062x; 1.5062x over previous
import functools

import jax
import jax.numpy as jnp
from jax import lax
from jax.experimental import pallas as pl
from jax.experimental.pallas import tpu as pltpu
from jax.experimental.pallas import tpu_sc as plsc

_VOCAB = 1000000
_DIM = 64
_BATCH = 16384
_HIST = 50
_N = _BATCH * _HIST
_NW = 32
_PER_W = _N // _NW
_CHUNK = 1024
_NCHUNK = _PER_W // _CHUNK
_SUB = 128
_NSUB = _CHUNK // _SUB
_GROUPS = _CHUNK // 16


def _dequant_group(rows_v, scl_v, out_v, g):
    ids = g * 16 + lax.iota(jnp.int32, 16)
    sw = scl_v[pl.ds(g * 16, 16)]
    sdup0 = plsc.bitcast((sw & 0xFFFF) | (sw << 16), jnp.float16)
    sdup1 = plsc.bitcast(lax.shift_right_logical(sw, 16)
                         | (sw & jnp.int32(-65536)), jnp.float16)
    c128 = jnp.full((32,), 128.0, jnp.float16)
    for w in range(16):
        v = plsc.load_gather(rows_v, [ids, jnp.full((16,), w, jnp.int32)])
        vx = v ^ jnp.int32(-2139062144)
        p01 = (vx & 0xFF) | ((vx & 0xFF00) << 8)
        p23 = ((lax.shift_right_logical(vx, 16) & 0xFF)
               | (lax.shift_right_logical(vx, 8) & 0xFF0000))
        s = sdup0 if w < 8 else sdup1
        f01 = (plsc.bitcast(p01, jnp.int16).astype(jnp.float16) - c128) * s
        f23 = (plsc.bitcast(p23, jnp.int16).astype(jnp.float16) - c128) * s
        plsc.store_scatter(out_v, [ids, jnp.full((16,), 2 * w, jnp.int32)],
                           plsc.bitcast(f01, jnp.int32))
        plsc.store_scatter(out_v, [ids, jnp.full((16,), 2 * w + 1, jnp.int32)],
                           plsc.bitcast(f23, jnp.int32))


def _sc_body(idx_hbm, w_hbm, s_hbm, out_hbm, idx_v, rows_v, scl_v, out_v,
             sem_w, sem_s):
    wid = lax.axis_index("s") * 2 + lax.axis_index("c")
    base_row = wid * _PER_W

    def chunk_body(c, _):
        r0 = pl.multiple_of(base_row + c * _CHUNK, _CHUNK)
        pltpu.sync_copy(idx_hbm.at[pl.ds(pl.multiple_of(r0 // _SUB, _NSUB), _NSUB)],
                        idx_v)
        copies = []
        for j in range(_NSUB):
            copies.append(pltpu.async_copy(
                w_hbm.at[idx_v.at[j]], rows_v.at[pl.ds(j * _SUB, _SUB)], sem_w))
            copies.append(pltpu.async_copy(
                s_hbm.at[idx_v.at[j]], scl_v.at[pl.ds(j * _SUB, _SUB)], sem_s))
        for cp in copies:
            cp.wait()

        def group_body(g, _):
            _dequant_group(rows_v, scl_v, out_v, g)
            return 0

        lax.fori_loop(0, _GROUPS, group_body, 0)
        pltpu.sync_copy(out_v, out_hbm.at[pl.ds(r0, _CHUNK)])
        return 0

    lax.fori_loop(0, _NCHUNK, chunk_body, 0)


@functools.partial(
    pl.kernel,
    out_type=jax.ShapeDtypeStruct((_N, 32), jnp.int32),
    mesh=plsc.VectorSubcoreMesh(core_axis_name="c", subcore_axis_name="s"),
    scratch_types=[
        pltpu.VMEM((_NSUB, _SUB), jnp.int32),
        pltpu.VMEM((_CHUNK, 16), jnp.int32),
        pltpu.VMEM((_CHUNK,), jnp.int32),
        pltpu.VMEM((_CHUNK, 32), jnp.int32),
        pltpu.SemaphoreType.DMA,
        pltpu.SemaphoreType.DMA,
    ],
    compiler_params=pltpu.CompilerParams(needs_layout_passes=False,
                                         use_tc_tiling_on_sc=False),
)
def _sc_lookup(idx_hbm, w_hbm, s_hbm, out_hbm, idx_v, rows_v, scl_v, out_v,
               sem_w, sem_s):
    _sc_body(idx_hbm, w_hbm, s_hbm, out_hbm, idx_v, rows_v, scl_v, out_v,
             sem_w, sem_s)


def kernel(indices, weight_int8, scales_fp16):
    V, D = weight_int8.shape
    idx2d = indices.reshape(_N // _SUB, _SUB)
    w8 = lax.bitcast_convert_type(weight_int8, jnp.uint8).astype(jnp.uint32)
    w_i32 = lax.bitcast_convert_type(
        w8[:, 0::4] | (w8[:, 1::4] << 8) | (w8[:, 2::4] << 16)
        | (w8[:, 3::4] << 24), jnp.int32)
    s16 = lax.bitcast_convert_type(scales_fp16, jnp.uint16).astype(jnp.uint32)
    s_i32 = lax.bitcast_convert_type(s16[:, 0] | (s16[:, 1] << 16), jnp.int32)
    out = _sc_lookup(idx2d, w_i32, s_i32)
    out = lax.bitcast_convert_type(out, jnp.float16)
    return out.reshape(indices.shape + (D,))


# speedup vs baseline: 2.4733x; 1.6421x over previous
import functools

import jax
import jax.numpy as jnp
from jax import lax
from jax.experimental import pallas as pl
from jax.experimental.pallas import tpu as pltpu
from jax.experimental.pallas import tpu_sc as plsc

_VOCAB = 1000000
_DIM = 64
_BATCH = 16384
_HIST = 50
_N = _BATCH * _HIST
_NW = 32
_PER_W = _N // _NW
_CHUNK = 1024
_NCHUNK = _PER_W // _CHUNK
_SUB = 128
_NSUB = _CHUNK // _SUB
_GROUPS = _CHUNK // 16


def _dequant_group(rows_v, scl_v, out_v, g):
    row_ids = g * 16 + lax.iota(jnp.int32, 16)
    sw = scl_v[pl.ds(g * 16, 16)]
    sdup0 = plsc.bitcast((sw & 0xFFFF) | (sw << 16), jnp.float16)
    sdup1 = plsc.bitcast(lax.shift_right_logical(sw, 16)
                         | (sw & jnp.int32(-65536)), jnp.float16)
    c128 = jnp.full((32,), 128.0, jnp.float16)
    for w in range(16):
        col = jnp.full((16,), w, jnp.int32)
        v = plsc.load_gather(rows_v, [row_ids, col])
        vx = v ^ jnp.int32(-2139062144)
        p01 = (vx & 0xFF) | ((vx & 0xFF00) << 8)
        p23 = ((lax.shift_right_logical(vx, 16) & 0xFF)
               | (lax.shift_right_logical(vx, 8) & 0xFF0000))
        s = sdup0 if w < 8 else sdup1
        f01 = (plsc.bitcast(p01, jnp.int16).astype(jnp.float16) - c128) * s
        f23 = (plsc.bitcast(p23, jnp.int16).astype(jnp.float16) - c128) * s
        plsc.store_scatter(out_v, [row_ids, jnp.full((16,), 2 * w, jnp.int32)],
                           plsc.bitcast(f01, jnp.int32))
        plsc.store_scatter(out_v, [row_ids, jnp.full((16,), 2 * w + 1, jnp.int32)],
                           plsc.bitcast(f23, jnp.int32))


def _sc_body(idx_hbm, w_hbm, s_hbm, out_hbm, idx_v, scl_v, rows_v, out_v,
             sem_w, sem_s):
    wid = lax.axis_index("s") * 2 + lax.axis_index("c")
    base_row = wid * _PER_W

    def chunk_body(c, _):
        r0 = pl.multiple_of(base_row + c * _CHUNK, _CHUNK)
        pltpu.sync_copy(idx_hbm.at[pl.ds(pl.multiple_of(r0 // _SUB, _NSUB), _NSUB)],
                        idx_v)
        copies = []
        for j in range(_NSUB):
            copies.append(pltpu.async_copy(
                w_hbm.at[idx_v.at[j]], rows_v.at[pl.ds(j * _SUB, _SUB)], sem_w))
            copies.append(pltpu.async_copy(
                s_hbm.at[idx_v.at[j]], scl_v.at[pl.ds(j * _SUB, _SUB)], sem_s))
        for cp in copies:
            cp.wait()

        def group_body(g, _):
            _dequant_group(rows_v, scl_v, out_v, g)
            return 0

        lax.fori_loop(0, _GROUPS, group_body, 0)
        pltpu.sync_copy(out_v, out_hbm.at[pl.ds(r0, _CHUNK)])
        return 0

    lax.fori_loop(0, _NCHUNK, chunk_body, 0)


@functools.partial(
    pl.kernel,
    out_type=jax.ShapeDtypeStruct((_N, 16 * 2), jnp.int32),
    mesh=plsc.VectorSubcoreMesh(core_axis_name="c", subcore_axis_name="s"),
    scratch_types=[
        pltpu.VMEM((_NSUB, _SUB), jnp.int32),
        pltpu.VMEM((_CHUNK,), jnp.int32),
        pltpu.VMEM((_CHUNK, 16), jnp.int32),
        pltpu.VMEM((_CHUNK, 32), jnp.int32),
        pltpu.SemaphoreType.DMA,
        pltpu.SemaphoreType.DMA,
    ],
    compiler_params=pltpu.CompilerParams(needs_layout_passes=False,
                                         use_tc_tiling_on_sc=False),
)
def _sc_lookup(idx_hbm, w_hbm, s_hbm, out_hbm, idx_v, scl_v, rows_v, out_v,
               sem_w, sem_s):
    _sc_body(idx_hbm, w_hbm, s_hbm, out_hbm, idx_v, scl_v, rows_v, out_v,
             sem_w, sem_s)


def kernel(indices, weight_int8, scales_fp16):
    V, D = weight_int8.shape
    idx2d = indices.reshape(_N // _SUB, _SUB)
    w_i32 = lax.bitcast_convert_type(weight_int8.reshape(V // 8, 128, 4),
                                     jnp.int32).reshape(V, D // 4)
    s_i32 = lax.bitcast_convert_type(scales_fp16, jnp.int32)
    out_i32 = _sc_lookup(idx2d, w_i32, s_i32)
    out = lax.bitcast_convert_type(out_i32, jnp.float16)
    return out.reshape(indices.shape + (D,))


# speedup vs baseline: 6.5630x; 2.6535x over previous
import functools

import jax
import jax.numpy as jnp
from jax import lax
from jax.experimental import pallas as pl
from jax.experimental.pallas import tpu as pltpu
from jax.experimental.pallas import tpu_sc as plsc

_VOCAB = 1000000
_DIM = 64
_BATCH = 16384
_HIST = 50
_N = _BATCH * _HIST
_NW = 32
_PER_W = _N // _NW
_CHUNK = 1024
_NCHUNK = _PER_W // _CHUNK
_SUB = 128
_NSUB = _CHUNK // _SUB
_GROUPS = _CHUNK // 16


def _dequant_group(rows_v, scl_v, out_v, g):
    row_ids = g * 16 + lax.iota(jnp.int32, 16)
    sw = scl_v[pl.ds(g * 16, 16)]
    sdup0 = plsc.bitcast((sw & 0xFFFF) | (sw << 16), jnp.float16)
    sdup1 = plsc.bitcast(lax.shift_right_logical(sw, 16)
                         | (sw & jnp.int32(-65536)), jnp.float16)
    c128 = jnp.full((32,), 128.0, jnp.float16)
    for w in range(16):
        col = jnp.full((16,), w, jnp.int32)
        v = plsc.load_gather(rows_v, [row_ids, col])
        vx = v ^ jnp.int32(-2139062144)
        p01 = (vx & 0xFF) | ((vx & 0xFF00) << 8)
        p23 = ((lax.shift_right_logical(vx, 16) & 0xFF)
               | (lax.shift_right_logical(vx, 8) & 0xFF0000))
        s = sdup0 if w < 8 else sdup1
        f01 = (plsc.bitcast(p01, jnp.int16).astype(jnp.float16) - c128) * s
        f23 = (plsc.bitcast(p23, jnp.int16).astype(jnp.float16) - c128) * s
        plsc.store_scatter(out_v, [row_ids, jnp.full((16,), 2 * w, jnp.int32)],
                           plsc.bitcast(f01, jnp.int32))
        plsc.store_scatter(out_v, [row_ids, jnp.full((16,), 2 * w + 1, jnp.int32)],
                           plsc.bitcast(f23, jnp.int32))


def _sc_body(idx_hbm, w_hbm, s_hbm, out_hbm, bufs):
    wid = lax.axis_index("s") * 2 + lax.axis_index("c")
    base_row = wid * _PER_W

    def fire(c, p):
        idx_v, scl_v, rows_v, _, sem_w, sem_s, _ = bufs[p]
        r0 = pl.multiple_of(base_row + c * _CHUNK, _CHUNK)
        pltpu.sync_copy(idx_hbm.at[pl.ds(pl.multiple_of(r0 // _SUB, _NSUB), _NSUB)],
                        idx_v)
        for j in range(_NSUB):
            pltpu.make_async_copy(
                w_hbm.at[idx_v.at[j]], rows_v.at[pl.ds(j * _SUB, _SUB)],
                sem_w).start()
            pltpu.make_async_copy(
                s_hbm.at[idx_v.at[j]], scl_v.at[pl.ds(j * _SUB, _SUB)],
                sem_s).start()

    def drain_compute_store(c, p):
        idx_v, scl_v, rows_v, out_v, sem_w, sem_s, sem_o = bufs[p]
        r0 = pl.multiple_of(base_row + c * _CHUNK, _CHUNK)
        for j in range(_NSUB):
            pltpu.make_async_copy(
                w_hbm.at[idx_v.at[j]], rows_v.at[pl.ds(j * _SUB, _SUB)],
                sem_w).wait()
            pltpu.make_async_copy(
                s_hbm.at[idx_v.at[j]], scl_v.at[pl.ds(j * _SUB, _SUB)],
                sem_s).wait()

        @pl.when(c >= 2)
        def _():
            pltpu.make_async_copy(
                out_v, out_hbm.at[pl.ds(r0, _CHUNK)], sem_o).wait()

        def group_body(g, _):
            _dequant_group(rows_v, scl_v, out_v, g)
            return 0

        lax.fori_loop(0, _GROUPS, group_body, 0)
        pltpu.make_async_copy(out_v, out_hbm.at[pl.ds(r0, _CHUNK)], sem_o).start()

    fire(0, 0)

    def chunk_body(c, _):
        @pl.when((c + 1 < _NCHUNK) & (c % 2 == 0))
        def _():
            fire(c + 1, 1)

        @pl.when((c + 1 < _NCHUNK) & (c % 2 == 1))
        def _():
            fire(c + 1, 0)

        @pl.when(c % 2 == 0)
        def _():
            drain_compute_store(c, 0)

        @pl.when(c % 2 == 1)
        def _():
            drain_compute_store(c, 1)

        return 0

    lax.fori_loop(0, _NCHUNK, chunk_body, 0)
    for p, c in ((1, _NCHUNK - 2), (0, _NCHUNK - 1)):
        _, _, _, out_v, _, _, sem_o = bufs[p]
        r0 = pl.multiple_of(base_row + c * _CHUNK, _CHUNK)
        pltpu.make_async_copy(out_v, out_hbm.at[pl.ds(r0, _CHUNK)], sem_o).wait()


@functools.partial(
    pl.kernel,
    out_type=jax.ShapeDtypeStruct((_N, 16 * 2), jnp.int32),
    mesh=plsc.VectorSubcoreMesh(core_axis_name="c", subcore_axis_name="s"),
    scratch_types=[
        pltpu.VMEM((_NSUB, _SUB), jnp.int32),
        pltpu.VMEM((_CHUNK,), jnp.int32),
        pltpu.VMEM((_CHUNK, 16), jnp.int32),
        pltpu.VMEM((_CHUNK, 32), jnp.int32),
        pltpu.SemaphoreType.DMA,
        pltpu.SemaphoreType.DMA,
        pltpu.SemaphoreType.DMA,
        pltpu.VMEM((_NSUB, _SUB), jnp.int32),
        pltpu.VMEM((_CHUNK,), jnp.int32),
        pltpu.VMEM((_CHUNK, 16), jnp.int32),
        pltpu.VMEM((_CHUNK, 32), jnp.int32),
        pltpu.SemaphoreType.DMA,
        pltpu.SemaphoreType.DMA,
        pltpu.SemaphoreType.DMA,
    ],
    compiler_params=pltpu.CompilerParams(needs_layout_passes=False,
                                         use_tc_tiling_on_sc=False),
)
def _sc_lookup(idx_hbm, w_hbm, s_hbm, out_hbm,
               idx0, scl0, rows0, out0, semw0, sems0, semo0,
               idx1, scl1, rows1, out1, semw1, sems1, semo1):
    bufs = [
        (idx0, scl0, rows0, out0, semw0, sems0, semo0),
        (idx1, scl1, rows1, out1, semw1, sems1, semo1),
    ]
    _sc_body(idx_hbm, w_hbm, s_hbm, out_hbm, bufs)


def kernel(indices, weight_int8, scales_fp16):
    V, D = weight_int8.shape
    idx2d = indices.reshape(_N // _SUB, _SUB)
    w_i32 = lax.bitcast_convert_type(weight_int8.reshape(V, D // 4, 4), jnp.int32)
    s_i32 = lax.bitcast_convert_type(scales_fp16, jnp.int32)
    out_i32 = _sc_lookup(idx2d, w_i32, s_i32)
    out = lax.bitcast_convert_type(out_i32, jnp.float16)
    return out.reshape(indices.shape + (D,))
